# Initial kernel scaffold; baseline (speedup 1.0000x reference)
import jax
import jax.numpy as jnp
from jax.experimental import pallas as pl


def kernel(x, edge_index, batch, params):
    raise NotImplementedError("write your pallas kernel here")


# speedup vs baseline: 2.6964x; 2.6964x over previous
import functools

import jax
import jax.numpy as jnp
from jax import lax
from jax.experimental import pallas as pl
from jax.experimental.pallas import tpu as pltpu
from jax.experimental.pallas import tpu_sc as plsc

N = 10000
NPAD = 10240
E = 320000
G = 16
D = 128

NTILES = 32
EPT = E // NTILES
CHUNK = 128
CPT = -(-EPT // CHUNK)
EPT_PAD = CPT * CHUNK
RPT = NPAD // 16
ZR = 64


def _agg_body(h_hbm, src_hbm, dst_hbm, out_hbm, sidx, didx, rows, zbuf, acc, sem):
    c = lax.axis_index("c")
    s = lax.axis_index("s")
    wid = s * 2 + c

    zero16 = jnp.zeros((16,), jnp.float32)

    def zset(i, carry):
        zbuf[i // 8, pl.ds((i % 8) * 16, 16)] = zero16
        return carry

    lax.fori_loop(0, ZR * 8, zset, 0)

    def zcp(i, carry):
        pltpu.sync_copy(zbuf, acc.at[pl.ds(s * RPT + i * ZR, ZR)])
        return carry

    lax.fori_loop(0, RPT // ZR, zcp, 0)
    plsc.subcore_barrier()

    def step(i, carry):
        base = wid * EPT_PAD + i * CHUNK
        pltpu.sync_copy(src_hbm.at[pl.ds(base, CHUNK)], sidx)
        pltpu.sync_copy(dst_hbm.at[pl.ds(base, CHUNK)], didx)
        pltpu.async_copy(h_hbm.at[sidx], rows, sem).wait()
        pltpu.sync_copy(rows, acc.at[didx], add=True)
        return carry

    lax.fori_loop(0, CPT, step, 0)
    plsc.subcore_barrier()

    pltpu.sync_copy(acc.at[pl.ds(s * RPT, RPT)],
                    out_hbm.at[c, pl.ds(s * RPT, RPT)])


@functools.cache
def _get_agg_sc():
    return pl.kernel(
        _agg_body,
        out_type=jax.ShapeDtypeStruct((2, NPAD, D), jnp.float32),
        mesh=plsc.VectorSubcoreMesh(core_axis_name="c", subcore_axis_name="s",
                                    num_cores=2, num_subcores=16),
        scratch_types=[
            pltpu.VMEM((CHUNK,), jnp.int32),
            pltpu.VMEM((CHUNK,), jnp.int32),
            pltpu.VMEM((CHUNK, D), jnp.float32),
            pltpu.VMEM((ZR, D), jnp.float32),
            pltpu.VMEM_SHARED((NPAD, D), jnp.float32),
            pltpu.SemaphoreType.DMA,
        ],
        name="gin_agg_sc",
    )


BLK = 256
NBLK = NPAD // BLK


def _row_mask(z):
    rows = pl.program_id(0) * BLK + lax.broadcasted_iota(jnp.int32, (BLK, 1), 0)
    return jnp.where(rows < N, z, 0.0)


def _mlp(t, w1, b1, w2, b2, final_relu):
    z = jnp.maximum(jnp.dot(t, w1, preferred_element_type=jnp.float32) + b1, 0.0)
    z = jnp.dot(z, w2, preferred_element_type=jnp.float32) + b2
    if final_relu:
        z = jnp.maximum(z, 0.0)
    return z


def _l1_body(h_ref, a_ref, wm1, bm1, wm2, bm2, wl1, bl1, wl2, bl2, o_ref):
    t = h_ref[...] + a_ref[0] + a_ref[1]
    zm = _mlp(t, wm1[...], bm1[...], wm2[...], bm2[...], True)
    zl = _mlp(t, wl1[...], bl1[...], wl2[...], bl2[...], True)
    o_ref[...] = _row_mask(jnp.concatenate([zm, zl], axis=1))


def _l2_body(h_ref, a_ref, wm1, bm1, wm2, bm2, wl1, bl1, wl2, bl2, om_ref, ol_ref):
    t = h_ref[...] + a_ref[0] + a_ref[1]
    om_ref[...] = _row_mask(_mlp(t[:, :64], wm1[...], bm1[...], wm2[...], bm2[...], True))
    ol_ref[...] = _row_mask(_mlp(t[:, 64:], wl1[...], bl1[...], wl2[...], bl2[...], True))


def _l34_body(final_relu, h_ref, a_ref, w1, b1, w2, b2, o_ref):
    t = h_ref[...] + a_ref[0] + a_ref[1]
    o_ref[...] = _row_mask(_mlp(t, w1[...], b1[...], w2[...], b2[...], final_relu))


def _run_mlp(body, h, aggs, weights, out_shapes):
    f = h.shape[1]
    in_specs = [
        pl.BlockSpec((BLK, f), lambda i: (i, 0)),
        pl.BlockSpec((2, BLK, f), lambda i: (0, i, 0)),
    ] + [pl.BlockSpec(w.shape, functools.partial(lambda nd, i: (0,) * nd, w.ndim))
         for w in weights]
    out_specs = [pl.BlockSpec((BLK, o.shape[1]), lambda i: (i, 0)) for o in out_shapes]
    if len(out_shapes) == 1:
        out_specs = out_specs[0]
        out_shape = out_shapes[0]
    else:
        out_shape = out_shapes
    return pl.pallas_call(
        body,
        grid=(NBLK,),
        in_specs=in_specs,
        out_specs=out_specs,
        out_shape=out_shape,
    )(h, aggs, *weights)


def _pool_body(hm_ref, hl_ref, b_ref, mu_ref, lv_ref):
    gids = lax.broadcasted_iota(jnp.int32, (G, NPAD), 0)
    onehot = (b_ref[...] == gids).astype(jnp.float32)
    counts = jnp.maximum(jnp.sum(onehot, axis=1, keepdims=True), 1.0)
    mu_ref[...] = jnp.dot(onehot, hm_ref[...], preferred_element_type=jnp.float32) / counts
    lv_ref[...] = jnp.dot(onehot, hl_ref[...], preferred_element_type=jnp.float32) / counts


def _dec_body(mu_ref, lv_ref, eps_ref, w1, b1, w2, b2, w3, b3, w4, b4, o_ref):
    std = jnp.exp(0.5 * lv_ref[...])
    z = mu_ref[...] + eps_ref[...] * std
    z = jnp.maximum(jnp.dot(z, w1[...], preferred_element_type=jnp.float32) + b1[...], 0.0)
    z = jnp.maximum(jnp.dot(z, w2[...], preferred_element_type=jnp.float32) + b2[...], 0.0)
    z = jnp.maximum(jnp.dot(z, w3[...], preferred_element_type=jnp.float32) + b3[...], 0.0)
    o_ref[...] = jax.nn.sigmoid(
        jnp.dot(z, w4[...], preferred_element_type=jnp.float32) + b4[...])


def _flat(mlp):
    w1, b1, w2, b2 = mlp
    return (w1, b1.reshape(1, -1), w2, b2.reshape(1, -1))


def kernel(x, edge_index, batch, params):
    x = x.astype(jnp.float32)
    src = edge_index[0].astype(jnp.int32)
    dst = edge_index[1].astype(jnp.int32)

    x_pad = jnp.pad(x, ((0, NPAD - N), (0, 0)))
    src_p = jnp.pad(src.reshape(NTILES, EPT), ((0, 0), (0, EPT_PAD - EPT)),
                    constant_values=NPAD - 1).reshape(-1)
    dst_p = jnp.pad(dst.reshape(NTILES, EPT), ((0, 0), (0, EPT_PAD - EPT)),
                    constant_values=NPAD - 1).reshape(-1)
    batch_p = jnp.pad(batch.astype(jnp.int32), (0, NPAD - N),
                      constant_values=G).reshape(1, NPAD)

    mu_w = [_flat(m) for m in params["mu"]]
    lg_w = [_flat(m) for m in params["log"]]
    agg_sc = _get_agg_sc()

    agg = agg_sc(x_pad, src_p, dst_p)
    h1 = _run_mlp(_l1_body, x_pad, agg, mu_w[0] + lg_w[0],
                  [jax.ShapeDtypeStruct((NPAD, 128), jnp.float32)])

    agg = agg_sc(h1, src_p, dst_p)
    h2m, h2l = _run_mlp(_l2_body, h1, agg, mu_w[1] + lg_w[1],
                        [jax.ShapeDtypeStruct((NPAD, 128), jnp.float32),
                         jax.ShapeDtypeStruct((NPAD, 128), jnp.float32)])

    l3 = functools.partial(_l34_body, True)
    l4 = functools.partial(_l34_body, False)
    h3m = _run_mlp(l3, h2m, agg_sc(h2m, src_p, dst_p), mu_w[2],
                   [jax.ShapeDtypeStruct((NPAD, 128), jnp.float32)])
    h3l = _run_mlp(l3, h2l, agg_sc(h2l, src_p, dst_p), lg_w[2],
                   [jax.ShapeDtypeStruct((NPAD, 128), jnp.float32)])
    h4m = _run_mlp(l4, h3m, agg_sc(h3m, src_p, dst_p), mu_w[3],
                   [jax.ShapeDtypeStruct((NPAD, 64), jnp.float32)])
    h4l = _run_mlp(l4, h3l, agg_sc(h3l, src_p, dst_p), lg_w[3],
                   [jax.ShapeDtypeStruct((NPAD, 64), jnp.float32)])

    mu, logvar = pl.pallas_call(
        _pool_body,
        out_shape=[jax.ShapeDtypeStruct((G, 64), jnp.float32),
                   jax.ShapeDtypeStruct((G, 64), jnp.float32)],
    )(h4m, h4l, batch_p)

    eps = jax.random.normal(jax.random.key(123), (G, 64), jnp.float32)
    fc = params["fc"]
    fw = []
    for w, b in fc:
        fw += [w, b.reshape(1, -1)]
    out = pl.pallas_call(
        _dec_body,
        out_shape=jax.ShapeDtypeStruct((G, fc[3][0].shape[1]), jnp.float32),
    )(mu, logvar, eps, *fw)

    return (out, mu, logvar)
